# baseline (speedup 1.0000x reference)
import jax
import jax.numpy as jnp
from jax.experimental import pallas as pl
from jax.experimental.pallas import tpu as pltpu

EPS = 1e-12
N_SKILLS = 8
RANK = 16
TS = 2048


def _router_body(task_ids_ref, ml_ref, w_ref):
    bsz = w_ref.shape[0]
    for b in range(bsz):
        tid = task_ids_ref[b]
        row = ml_ref[pl.ds(tid, 1), :]
        p = jax.nn.sigmoid(row)
        w_ref[pl.ds(b, 1), :] = p / (jnp.sum(p) + EPS)


def _fused_body(wts_ref, x_ref, w_ref, bias_ref, la_ref, lb_ref, out_ref,
                weff_ref):
    b = pl.program_id(0)
    o = pl.program_id(1)
    n_o = pl.num_programs(1)
    oc = out_ref.shape[2]

    @pl.when(o == 0)
    def _build_weff():
        A = la_ref[0] * wts_ref[b, 0]
        Bm = lb_ref[0] * wts_ref[b, 0]
        for k in range(1, N_SKILLS):
            A = A + la_ref[k] * wts_ref[b, k]
            Bm = Bm + lb_ref[k] * wts_ref[b, k]
        abT = jax.lax.dot_general(
            Bm, A, (((0,), (1,)), ((), ())),
            preferred_element_type=jnp.float32)
        weff_ref[...] = w_ref[...] + abT * (1.0 / RANK)

    out_ref[0] = jax.lax.dot_general(
        x_ref[0], weff_ref[pl.ds(o * oc, oc), :], (((1,), (1,)), ((), ())),
        preferred_element_type=jnp.float32) + bias_ref[...]


def kernel(x, task_ids, module_logits, weight, bias, lora_a, lora_b):
    bsz, seq, in_f = x.shape
    out_f = weight.shape[0]
    n_tasks, n_sk = module_logits.shape

    wts = pl.pallas_call(
        _router_body,
        in_specs=[
            pl.BlockSpec(memory_space=pltpu.SMEM),
            pl.BlockSpec(memory_space=pltpu.VMEM),
        ],
        out_specs=pl.BlockSpec(memory_space=pltpu.VMEM),
        out_shape=jax.ShapeDtypeStruct((bsz, n_sk), jnp.float32),
    )(task_ids.astype(jnp.int32), module_logits)

    bias2 = bias.reshape(1, out_f)
    la = lora_a.reshape(n_sk, in_f, RANK)
    lb = lora_b.reshape(n_sk, RANK, out_f)

    n_o = 2
    oc = out_f // n_o
    out = pl.pallas_call(
        _fused_body,
        grid=(bsz, n_o),
        in_specs=[
            pl.BlockSpec(memory_space=pltpu.SMEM),
            pl.BlockSpec((1, TS, in_f), lambda b, o: (b, 0, 0)),
            pl.BlockSpec((out_f, in_f), lambda b, o: (0, 0)),
            pl.BlockSpec((1, oc), lambda b, o: (0, o)),
            pl.BlockSpec((n_sk, in_f, RANK), lambda b, o: (0, 0, 0)),
            pl.BlockSpec((n_sk, RANK, out_f), lambda b, o: (0, 0, 0)),
        ],
        out_specs=pl.BlockSpec((1, TS, oc), lambda b, o: (b, 0, o)),
        out_shape=jax.ShapeDtypeStruct((bsz, seq, out_f), jnp.float32),
        scratch_shapes=[pltpu.VMEM((out_f, in_f), jnp.float32)],
        compiler_params=pltpu.CompilerParams(
            dimension_semantics=("parallel", "arbitrary")),
    )(wts, x, weight, bias2, la, lb)
    return out


# speedup vs baseline: 1.2957x; 1.2957x over previous
import jax
import jax.numpy as jnp
from jax.experimental import pallas as pl
from jax.experimental.pallas import tpu as pltpu

EPS = 1e-12
N_SKILLS = 8
RANK = 16
NCHUNK = 4


def _router_body(task_ids_ref, ml_ref, w_ref):
    bsz = w_ref.shape[0]
    for b in range(bsz):
        tid = task_ids_ref[b]
        row = ml_ref[pl.ds(tid, 1), :]
        p = jax.nn.sigmoid(row)
        w_ref[pl.ds(b, 1), :] = p / (jnp.sum(p) + EPS)


def _fused_body(wts_ref, x_ref, w_ref, bias_ref, la_ref, lb_ref, out_ref,
                xbuf, obuf, weff_ref, lsem, ssem):
    bsz, seq, _ = x_ref.shape
    ck = seq // NCHUNK

    def load(b):
        slot = b % 2
        for c in range(NCHUNK):
            pltpu.make_async_copy(
                x_ref.at[b, pl.ds(c * ck, ck)],
                xbuf.at[slot, pl.ds(c * ck, ck)], lsem.at[slot, c]).start()

    def load_wait(b):
        slot = b % 2
        for c in range(NCHUNK):
            pltpu.make_async_copy(
                x_ref.at[b, pl.ds(c * ck, ck)],
                xbuf.at[slot, pl.ds(c * ck, ck)], lsem.at[slot, c]).wait()

    def store(b):
        slot = b % 2
        for c in range(NCHUNK):
            pltpu.make_async_copy(
                obuf.at[slot, pl.ds(c * ck, ck)],
                out_ref.at[b, pl.ds(c * ck, ck)], ssem.at[slot, c]).start()

    def store_wait(b):
        slot = b % 2
        for c in range(NCHUNK):
            pltpu.make_async_copy(
                obuf.at[slot, pl.ds(c * ck, ck)],
                out_ref.at[b, pl.ds(c * ck, ck)], ssem.at[slot, c]).wait()

    load(0)
    for b in range(bsz):
        if b + 1 < bsz:
            load(b + 1)
        A = la_ref[0] * wts_ref[b, 0]
        Bm = lb_ref[0] * wts_ref[b, 0]
        for k in range(1, N_SKILLS):
            A = A + la_ref[k] * wts_ref[b, k]
            Bm = Bm + lb_ref[k] * wts_ref[b, k]
        abT = jax.lax.dot_general(
            Bm, A, (((0,), (1,)), ((), ())),
            preferred_element_type=jnp.float32)
        weff_ref[...] = w_ref[...] + abT * (1.0 / RANK)

        load_wait(b)
        if b >= 2:
            store_wait(b - 2)
        obuf[b % 2] = jax.lax.dot_general(
            xbuf[b % 2], weff_ref[...], (((1,), (1,)), ((), ())),
            preferred_element_type=jnp.float32) + bias_ref[...]
        store(b)
    for b in range(max(0, bsz - 2), bsz):
        store_wait(b)


def kernel(x, task_ids, module_logits, weight, bias, lora_a, lora_b):
    bsz, seq, in_f = x.shape
    out_f = weight.shape[0]
    n_tasks, n_sk = module_logits.shape

    wts = pl.pallas_call(
        _router_body,
        in_specs=[
            pl.BlockSpec(memory_space=pltpu.SMEM),
            pl.BlockSpec(memory_space=pltpu.VMEM),
        ],
        out_specs=pl.BlockSpec(memory_space=pltpu.VMEM),
        out_shape=jax.ShapeDtypeStruct((bsz, n_sk), jnp.float32),
    )(task_ids.astype(jnp.int32), module_logits)

    bias2 = bias.reshape(1, out_f)
    la = lora_a.reshape(n_sk, in_f, RANK)
    lb = lora_b.reshape(n_sk, RANK, out_f)

    out = pl.pallas_call(
        _fused_body,
        in_specs=[
            pl.BlockSpec(memory_space=pltpu.SMEM),
            pl.BlockSpec(memory_space=pl.ANY),
            pl.BlockSpec(memory_space=pltpu.VMEM),
            pl.BlockSpec(memory_space=pltpu.VMEM),
            pl.BlockSpec(memory_space=pltpu.VMEM),
            pl.BlockSpec(memory_space=pltpu.VMEM),
        ],
        out_specs=pl.BlockSpec(memory_space=pl.ANY),
        out_shape=jax.ShapeDtypeStruct((bsz, seq, out_f), jnp.float32),
        scratch_shapes=[
            pltpu.VMEM((2, seq, in_f), jnp.float32),
            pltpu.VMEM((2, seq, out_f), jnp.float32),
            pltpu.VMEM((out_f, in_f), jnp.float32),
            pltpu.SemaphoreType.DMA((2, NCHUNK)),
            pltpu.SemaphoreType.DMA((2, NCHUNK)),
        ],
    )(wts, x, weight, bias2, la, lb)
    return out


# speedup vs baseline: 1.3094x; 1.0106x over previous
import jax
import jax.numpy as jnp
from jax.experimental import pallas as pl
from jax.experimental.pallas import tpu as pltpu

EPS = 1e-12
N_SKILLS = 8
RANK = 16
NCHUNK = 4


def _fused_body(tid_ref, ml_ref, x_ref, w_ref, bias_ref, la_ref, lb_ref,
                out_ref, xbuf, obuf, weff_ref, lsem, ssem):
    bsz, seq, _ = x_ref.shape
    ck = seq // NCHUNK

    def load(b):
        slot = b % 2
        for c in range(NCHUNK):
            pltpu.make_async_copy(
                x_ref.at[b, pl.ds(c * ck, ck)],
                xbuf.at[slot, pl.ds(c * ck, ck)], lsem.at[slot, c]).start()

    def load_wait(b):
        slot = b % 2
        for c in range(NCHUNK):
            pltpu.make_async_copy(
                x_ref.at[b, pl.ds(c * ck, ck)],
                xbuf.at[slot, pl.ds(c * ck, ck)], lsem.at[slot, c]).wait()

    def store(b):
        slot = b % 2
        for c in range(NCHUNK):
            pltpu.make_async_copy(
                obuf.at[slot, pl.ds(c * ck, ck)],
                out_ref.at[b, pl.ds(c * ck, ck)], ssem.at[slot, c]).start()

    def store_wait(b):
        slot = b % 2
        for c in range(NCHUNK):
            pltpu.make_async_copy(
                obuf.at[slot, pl.ds(c * ck, ck)],
                out_ref.at[b, pl.ds(c * ck, ck)], ssem.at[slot, c]).wait()

    load(0)
    for b in range(bsz):
        if b + 1 < bsz:
            load(b + 1)
        tid = tid_ref[b]
        sig = [1.0 / (1.0 + jnp.exp(-ml_ref[tid, k])) for k in range(N_SKILLS)]
        tot = sig[0]
        for k in range(1, N_SKILLS):
            tot = tot + sig[k]
        inv = 1.0 / (tot + EPS)
        wt = [s * inv for s in sig]
        A = la_ref[0] * wt[0]
        Bm = lb_ref[0] * wt[0]
        for k in range(1, N_SKILLS):
            A = A + la_ref[k] * wt[k]
            Bm = Bm + lb_ref[k] * wt[k]
        abT = jax.lax.dot_general(
            Bm, A, (((0,), (1,)), ((), ())),
            preferred_element_type=jnp.float32)
        weff_ref[...] = w_ref[...] + abT * (1.0 / RANK)

        load_wait(b)
        if b >= 2:
            store_wait(b - 2)
        obuf[b % 2] = jax.lax.dot_general(
            xbuf[b % 2], weff_ref[...], (((1,), (1,)), ((), ())),
            preferred_element_type=jnp.float32) + bias_ref[...]
        store(b)
    for b in range(max(0, bsz - 2), bsz):
        store_wait(b)


def kernel(x, task_ids, module_logits, weight, bias, lora_a, lora_b):
    bsz, seq, in_f = x.shape
    out_f = weight.shape[0]
    n_tasks, n_sk = module_logits.shape

    bias2 = bias.reshape(1, out_f)
    la = lora_a.reshape(n_sk, in_f, RANK)
    lb = lora_b.reshape(n_sk, RANK, out_f)

    out = pl.pallas_call(
        _fused_body,
        in_specs=[
            pl.BlockSpec(memory_space=pltpu.SMEM),
            pl.BlockSpec(memory_space=pltpu.SMEM),
            pl.BlockSpec(memory_space=pl.ANY),
            pl.BlockSpec(memory_space=pltpu.VMEM),
            pl.BlockSpec(memory_space=pltpu.VMEM),
            pl.BlockSpec(memory_space=pltpu.VMEM),
            pl.BlockSpec(memory_space=pltpu.VMEM),
        ],
        out_specs=pl.BlockSpec(memory_space=pl.ANY),
        out_shape=jax.ShapeDtypeStruct((bsz, seq, out_f), jnp.float32),
        scratch_shapes=[
            pltpu.VMEM((2, seq, in_f), jnp.float32),
            pltpu.VMEM((2, seq, out_f), jnp.float32),
            pltpu.VMEM((out_f, in_f), jnp.float32),
            pltpu.SemaphoreType.DMA((2, NCHUNK)),
            pltpu.SemaphoreType.DMA((2, NCHUNK)),
        ],
    )(task_ids.astype(jnp.int32), module_logits, x, weight, bias2, la, lb)
    return out


# speedup vs baseline: 1.3434x; 1.0260x over previous
import jax
import jax.numpy as jnp
from jax.experimental import pallas as pl
from jax.experimental.pallas import tpu as pltpu

EPS = 1e-12
N_SKILLS = 8
RANK = 16
NCHUNK = 4


def _fused_body(tid_ref, ml_ref, x_ref, w_ref, bias_ref, la_ref, lb_ref,
                out_ref, xbuf, obuf, weff_ref, lsem, ssem):
    bsz, seq, _ = x_ref.shape
    ck = seq // NCHUNK

    def load_chunk(b, c):
        pltpu.make_async_copy(
            x_ref.at[b, pl.ds(c * ck, ck)],
            xbuf.at[b % 2, pl.ds(c * ck, ck)], lsem.at[b % 2, c]).start()

    def load_wait_chunk(b, c):
        pltpu.make_async_copy(
            x_ref.at[b, pl.ds(c * ck, ck)],
            xbuf.at[b % 2, pl.ds(c * ck, ck)], lsem.at[b % 2, c]).wait()

    def store_chunk(b, c):
        pltpu.make_async_copy(
            obuf.at[b % 2, pl.ds(c * ck, ck)],
            out_ref.at[b, pl.ds(c * ck, ck)], ssem.at[b % 2, c]).start()

    def store_wait_chunk(b, c):
        pltpu.make_async_copy(
            obuf.at[b % 2, pl.ds(c * ck, ck)],
            out_ref.at[b, pl.ds(c * ck, ck)], ssem.at[b % 2, c]).wait()

    for c in range(NCHUNK):
        load_chunk(0, c)
    for b in range(bsz):
        if b + 1 < bsz:
            for c in range(NCHUNK):
                load_chunk(b + 1, c)
        tid = tid_ref[b]
        sig = [1.0 / (1.0 + jnp.exp(-ml_ref[tid, k])) for k in range(N_SKILLS)]
        tot = sig[0]
        for k in range(1, N_SKILLS):
            tot = tot + sig[k]
        inv = 1.0 / (tot + EPS)
        wt = [s * inv for s in sig]
        A = la_ref[0] * wt[0]
        Bm = lb_ref[0] * wt[0]
        for k in range(1, N_SKILLS):
            A = A + la_ref[k] * wt[k]
            Bm = Bm + lb_ref[k] * wt[k]
        abT = jax.lax.dot_general(
            Bm, A, (((0,), (1,)), ((), ())),
            preferred_element_type=jnp.float32)
        weff_ref[...] = w_ref[...] + abT * (1.0 / RANK)

        for c in range(NCHUNK):
            load_wait_chunk(b, c)
            if b >= 2:
                store_wait_chunk(b - 2, c)
            obuf[b % 2, pl.ds(c * ck, ck)] = jax.lax.dot_general(
                xbuf[b % 2, pl.ds(c * ck, ck)], weff_ref[...],
                (((1,), (1,)), ((), ())),
                preferred_element_type=jnp.float32) + bias_ref[...]
            store_chunk(b, c)
    for b in range(max(0, bsz - 2), bsz):
        for c in range(NCHUNK):
            store_wait_chunk(b, c)


def kernel(x, task_ids, module_logits, weight, bias, lora_a, lora_b):
    bsz, seq, in_f = x.shape
    out_f = weight.shape[0]
    n_tasks, n_sk = module_logits.shape

    bias2 = bias.reshape(1, out_f)
    la = lora_a.reshape(n_sk, in_f, RANK)
    lb = lora_b.reshape(n_sk, RANK, out_f)

    out = pl.pallas_call(
        _fused_body,
        in_specs=[
            pl.BlockSpec(memory_space=pltpu.SMEM),
            pl.BlockSpec(memory_space=pltpu.SMEM),
            pl.BlockSpec(memory_space=pl.ANY),
            pl.BlockSpec(memory_space=pltpu.VMEM),
            pl.BlockSpec(memory_space=pltpu.VMEM),
            pl.BlockSpec(memory_space=pltpu.VMEM),
            pl.BlockSpec(memory_space=pltpu.VMEM),
        ],
        out_specs=pl.BlockSpec(memory_space=pl.ANY),
        out_shape=jax.ShapeDtypeStruct((bsz, seq, out_f), jnp.float32),
        scratch_shapes=[
            pltpu.VMEM((2, seq, in_f), jnp.float32),
            pltpu.VMEM((2, seq, out_f), jnp.float32),
            pltpu.VMEM((out_f, in_f), jnp.float32),
            pltpu.SemaphoreType.DMA((2, NCHUNK)),
            pltpu.SemaphoreType.DMA((2, NCHUNK)),
        ],
    )(task_ids.astype(jnp.int32), module_logits, x, weight, bias2, la, lb)
    return out
